# baseline (speedup 1.0000x reference)
import functools

import jax
import jax.numpy as jnp
from jax import lax
from jax.experimental import pallas as pl
from jax.experimental.pallas import tpu as pltpu
from jax.experimental.pallas import tpu_sc as plsc

N = 10000
E = 320000
D = 128
C = 2

NC = 2
NS = 16
NW = NC * NS
EPW = E // NW
K = 80
NCHUNK = EPW // K
NP = 10240
ROWS_PT = NP // NS

_f32 = jnp.float32


_RB = 2000


def _proj_body(x_ref, w_ref, a2_ref, h_ref, sa_ref):
    h = jnp.dot(x_ref[...], w_ref[...], preferred_element_type=_f32)
    h_ref[...] = h
    sa_ref[...] = jnp.dot(h, a2_ref[...], preferred_element_type=_f32)


def _tc_proj(x, W, a2):
    return pl.pallas_call(
        _proj_body,
        grid=(N // _RB,),
        in_specs=[
            pl.BlockSpec((_RB, D), lambda i: (i, 0)),
            pl.BlockSpec((D, D), lambda i: (0, 0)),
            pl.BlockSpec((D, 2), lambda i: (0, 0)),
        ],
        out_specs=[
            pl.BlockSpec((_RB, D), lambda i: (i, 0)),
            pl.BlockSpec((_RB, 2), lambda i: (i, 0)),
        ],
        out_shape=[
            jax.ShapeDtypeStruct((N, D), _f32),
            jax.ShapeDtypeStruct((N, 2), _f32),
        ],
    )(x, W, a2)


def _combine_proj_body(acc_ref, den_ref, b_ref, w_ref, a2_ref, h_ref, sa_ref):
    dtot = jnp.sum(den_ref[...], axis=1, keepdims=True) + 1e-16
    inv = 1.0 / dtot
    out = (acc_ref[0] + acc_ref[1]) * inv + b_ref[...]
    out = jnp.maximum(out, 0.0)
    h = jnp.dot(out, w_ref[...], preferred_element_type=_f32)
    h_ref[...] = h
    sa_ref[...] = jnp.dot(h, a2_ref[...], preferred_element_type=_f32)


def _tc_combine_proj(acc, den, b, W, a2):
    return pl.pallas_call(
        _combine_proj_body,
        grid=(N // _RB,),
        in_specs=[
            pl.BlockSpec((NC, _RB, D), lambda i: (0, i, 0)),
            pl.BlockSpec((_RB, NC), lambda i: (i, 0)),
            pl.BlockSpec((1, D), lambda i: (0, 0)),
            pl.BlockSpec((D, D), lambda i: (0, 0)),
            pl.BlockSpec((D, 2), lambda i: (0, 0)),
        ],
        out_specs=[
            pl.BlockSpec((_RB, D), lambda i: (i, 0)),
            pl.BlockSpec((_RB, 2), lambda i: (i, 0)),
        ],
        out_shape=[
            jax.ShapeDtypeStruct((N, D), _f32),
            jax.ShapeDtypeStruct((N, 2), _f32),
        ],
    )(acc, den, b, W, a2)


def _classifier_body(acc_ref, den_ref, b_ref, wc_ref, bc_ref, out_ref):
    dtot = jnp.sum(den_ref[...], axis=1, keepdims=True) + 1e-16
    inv = 1.0 / dtot
    h = (acc_ref[0] + acc_ref[1]) * inv + b_ref[...]
    h = jnp.maximum(h, 0.0)
    logits = jnp.dot(h, wc_ref[...], preferred_element_type=_f32) + bc_ref[...]
    m = jnp.max(logits, axis=1, keepdims=True)
    lse = m + jnp.log(jnp.sum(jnp.exp(logits - m), axis=1, keepdims=True))
    out_ref[...] = logits - lse


def _tc_classifier(acc, den, b, Wc, bc):
    return pl.pallas_call(
        _classifier_body,
        grid=(N // _RB,),
        in_specs=[
            pl.BlockSpec((NC, _RB, D), lambda i: (0, i, 0)),
            pl.BlockSpec((_RB, NC), lambda i: (i, 0)),
            pl.BlockSpec((1, D), lambda i: (0, 0)),
            pl.BlockSpec((D, C), lambda i: (0, 0)),
            pl.BlockSpec((1, C), lambda i: (0, 0)),
        ],
        out_specs=pl.BlockSpec((_RB, C), lambda i: (i, 0)),
        out_shape=jax.ShapeDtypeStruct((N, C), _f32),
    )(acc, den, b, Wc, bc)


_MESH = plsc.VectorSubcoreMesh(
    core_axis_name="c", subcore_axis_name="s", num_cores=NC, num_subcores=NS
)


NGRP = (NCHUNK - 1) // 4
ZR = NP // NS


@functools.partial(
    pl.kernel,
    mesh=_MESH,
    compiler_params=pltpu.CompilerParams(needs_layout_passes=False),
    out_type=[
        jax.ShapeDtypeStruct((NC, NP, D), _f32),
        jax.ShapeDtypeStruct((NC, 1, NP), _f32),
    ],
    scratch_types=[
        [pltpu.VMEM((K,), jnp.int32)] * 4,
        [pltpu.VMEM((K,), jnp.int32)] * 4,
        [pltpu.VMEM((K,), _f32)] * 4,
        [pltpu.VMEM((K,), _f32)] * 4,
        [pltpu.VMEM((K,), _f32)] * 2,
        [pltpu.VMEM((K, D), _f32)] * 4,
        pltpu.VMEM((ZR,), _f32),
        pltpu.VMEM_SHARED((NP, D), _f32),
        pltpu.VMEM_SHARED((NP,), _f32),
        pltpu.VMEM_SHARED((N,), _f32),
        pltpu.VMEM_SHARED((N,), _f32),
        [pltpu.SemaphoreType.DMA] * 4,
        [pltpu.SemaphoreType.DMA] * 4,
        [pltpu.SemaphoreType.DMA] * 4,
        [pltpu.SemaphoreType.DMA] * 4,
        [pltpu.SemaphoreType.DMA] * 4,
        [pltpu.SemaphoreType.DMA] * 2,
        [pltpu.SemaphoreType.DMA] * 4,
    ],
)
def _sc_edges(src_hbm, dst_hbm, as_hbm, ad_hbm, h_hbm, acc_hbm, den_hbm,
              sbufs, dbufs, pav, pdv, pvs, bufs, zbuf, acc_sh, den_sh,
              as_sh, ad_sh, si, sd, spa, spd, sg, sp, ss):
    cid = lax.axis_index("c")
    sid = lax.axis_index("s")
    wid = sid * NC + cid

    for t in range(ZR // 16):
        zbuf[pl.ds(t * 16, 16)] = jnp.zeros((16,), _f32)

    def _zrow(i, carry):
        for j in range(D // 16):
            bufs[0][i, pl.ds(j * 16, 16)] = jnp.zeros((16,), _f32)
        return carry
    lax.fori_loop(0, K, _zrow, 0)

    rbase = pl.multiple_of(sid * ROWS_PT, 8)
    for t in range(ROWS_PT // K):
        pltpu.sync_copy(bufs[0], acc_sh.at[pl.ds(rbase + t * K, K)])
    pltpu.sync_copy(zbuf, den_sh.at[pl.ds(rbase, ZR)])

    @pl.when(sid == 0)
    def _():
        pltpu.sync_copy(as_hbm, as_sh)

    @pl.when(sid == 1)
    def _():
        pltpu.sync_copy(ad_hbm, ad_sh)
    plsc.subcore_barrier()

    def _issue_idx(c, slot):
        pltpu.async_copy(src_hbm.at[wid, c, 0], sbufs[slot], si[slot])
        pltpu.async_copy(dst_hbm.at[wid, c, 0], dbufs[slot], sd[slot])

    def _wait(src, dst, sem):
        pltpu.make_async_copy(src, dst, sem).wait()

    def _issue_gathers(slot):
        pltpu.async_copy(h_hbm.at[sbufs[slot]], bufs[slot], sg[slot])
        pltpu.async_copy(as_sh.at[sbufs[slot]], pav[slot], spa[slot])
        pltpu.async_copy(ad_sh.at[dbufs[slot]], pdv[slot], spd[slot])

    def _do_chunk(c, b, in_loop):
        rb = b % 2
        nslot = (b + 2) % 4
        gslot = (b + 1) % 4

        @pl.when(c >= 2)
        def _():
            _wait(pvs[rb], den_sh.at[dbufs[nslot]], sp[rb])
            _wait(bufs[nslot], acc_sh.at[dbufs[0]], ss[nslot])
        if in_loop:
            @pl.when(c + 2 < NCHUNK)
            def _():
                _issue_idx(c + 2, nslot)

            _wait(src_hbm.at[wid, 0, 0], sbufs[gslot], si[gslot])
            _wait(dst_hbm.at[wid, 0, 0], dbufs[gslot], sd[gslot])
            _issue_gathers(gslot)

        _wait(as_sh.at[sbufs[0]], pav[b], spa[b])
        _wait(ad_sh.at[dbufs[0]], pdv[b], spd[b])
        for j in range(K // 16):
            e = pav[b][pl.ds(j * 16, 16)] + pdv[b][pl.ds(j * 16, 16)]
            e = jnp.where(e >= 0.0, e, 0.2 * e)
            pvs[rb][pl.ds(j * 16, 16)] = jnp.exp(e)
        pltpu.async_copy(pvs[rb], den_sh.at[dbufs[b]], sp[rb], add=True)

        _wait(h_hbm.at[sbufs[b]], bufs[b], sg[b])

        def _scale(t, c2):
            p16 = pvs[rb][pl.ds(t * 16, 16)]
            for l in range(16):
                i = t * 16 + l
                ps = p16[l]
                for j in range(D // 16):
                    bufs[b][i, pl.ds(j * 16, 16)] = (
                        bufs[b][i, pl.ds(j * 16, 16)] * ps)
            return c2
        lax.fori_loop(0, K // 16, _scale, 0)

        pltpu.async_copy(bufs[b], acc_sh.at[dbufs[b]], ss[b], add=True)

    _issue_idx(0, 0)
    _issue_idx(1, 1)
    _wait(src_hbm.at[wid, 0, 0], sbufs[0], si[0])
    _wait(dst_hbm.at[wid, 0, 0], dbufs[0], sd[0])
    _issue_gathers(0)

    def _group(g, carry):
        for b in range(4):
            _do_chunk(g * 4 + b, b, True)
        return carry

    lax.fori_loop(0, NGRP, _group, 0)
    _do_chunk(NCHUNK - 1, 0, False)

    _wait(bufs[3], acc_sh.at[dbufs[0]], ss[3])
    _wait(bufs[0], acc_sh.at[dbufs[0]], ss[0])
    for rb in range(2):
        _wait(pvs[rb], den_sh.at[dbufs[0]], sp[rb])

    plsc.subcore_barrier()
    pltpu.sync_copy(den_sh.at[pl.ds(rbase, ZR)],
                    den_hbm.at[cid, 0, pl.ds(rbase, ZR)])
    pltpu.sync_copy(acc_sh.at[pl.ds(rbase, ROWS_PT)],
                    acc_hbm.at[cid, pl.ds(rbase, ROWS_PT)])


def kernel(x, edge_index, W1, a_src1, a_dst1, b1, W2, a_src2, a_dst2, b2,
           Wc, bc):
    src = edge_index[0].reshape(NW, NCHUNK, 1, K)
    dst = edge_index[1].reshape(NW, NCHUNK, 1, K)
    a21 = jnp.stack([a_src1, a_dst1], axis=1)
    a22 = jnp.stack([a_src2, a_dst2], axis=1)

    h1, sa1 = _tc_proj(x, W1, a21)
    acc1, den1 = _sc_edges(src, dst, sa1[:, 0], sa1[:, 1], h1)
    h2, sa2 = _tc_combine_proj(acc1, den1.reshape(NC, NP)[:, :N].T,
                               b1.reshape(1, D), W2, a22)
    acc2, den2 = _sc_edges(src, dst, sa2[:, 0], sa2[:, 1], h2)
    return _tc_classifier(acc2, den2.reshape(NC, NP)[:, :N].T,
                          b2.reshape(1, D), Wc, bc.reshape(1, C))


# speedup vs baseline: 1.0484x; 1.0484x over previous
import functools

import jax
import jax.numpy as jnp
from jax import lax
from jax.experimental import pallas as pl
from jax.experimental.pallas import tpu as pltpu
from jax.experimental.pallas import tpu_sc as plsc

N = 10000
E = 320000
D = 128
C = 2

NC = 2
NS = 16
NW = NC * NS
EPW = E // NW
K = 80
NCHUNK = EPW // K
NP = 10240
ROWS_PT = NP // NS

_f32 = jnp.float32


_RB = 2000


def _proj_body(x_ref, w_ref, a2_ref, h_ref, sa_ref):
    h = jnp.dot(x_ref[...], w_ref[...], preferred_element_type=_f32)
    h_ref[...] = h
    sa_ref[...] = jnp.dot(h, a2_ref[...], preferred_element_type=_f32)


def _tc_proj(x, W, a2):
    return pl.pallas_call(
        _proj_body,
        grid=(N // _RB,),
        in_specs=[
            pl.BlockSpec((_RB, D), lambda i: (i, 0)),
            pl.BlockSpec((D, D), lambda i: (0, 0)),
            pl.BlockSpec((D, 2), lambda i: (0, 0)),
        ],
        out_specs=[
            pl.BlockSpec((_RB, D), lambda i: (i, 0)),
            pl.BlockSpec((_RB, 2), lambda i: (i, 0)),
        ],
        out_shape=[
            jax.ShapeDtypeStruct((N, D), _f32),
            jax.ShapeDtypeStruct((N, 2), _f32),
        ],
    )(x, W, a2)


def _combine_proj_body(acc_ref, den_ref, b_ref, w_ref, a2_ref, h_ref, sa_ref):
    dtot = jnp.sum(den_ref[...], axis=1, keepdims=True) + 1e-16
    inv = 1.0 / dtot
    out = (acc_ref[0] + acc_ref[1]) * inv + b_ref[...]
    out = jnp.maximum(out, 0.0)
    h = jnp.dot(out, w_ref[...], preferred_element_type=_f32)
    h_ref[...] = h
    sa_ref[...] = jnp.dot(h, a2_ref[...], preferred_element_type=_f32)


def _tc_combine_proj(acc, den, b, W, a2):
    return pl.pallas_call(
        _combine_proj_body,
        grid=(N // _RB,),
        in_specs=[
            pl.BlockSpec((NC, _RB, D), lambda i: (0, i, 0)),
            pl.BlockSpec((_RB, NC), lambda i: (i, 0)),
            pl.BlockSpec((1, D), lambda i: (0, 0)),
            pl.BlockSpec((D, D), lambda i: (0, 0)),
            pl.BlockSpec((D, 2), lambda i: (0, 0)),
        ],
        out_specs=[
            pl.BlockSpec((_RB, D), lambda i: (i, 0)),
            pl.BlockSpec((_RB, 2), lambda i: (i, 0)),
        ],
        out_shape=[
            jax.ShapeDtypeStruct((N, D), _f32),
            jax.ShapeDtypeStruct((N, 2), _f32),
        ],
    )(acc, den, b, W, a2)


def _classifier_body(acc_ref, den_ref, b_ref, wc_ref, bc_ref, out_ref):
    dtot = jnp.sum(den_ref[...], axis=1, keepdims=True) + 1e-16
    inv = 1.0 / dtot
    h = (acc_ref[0] + acc_ref[1]) * inv + b_ref[...]
    h = jnp.maximum(h, 0.0)
    logits = jnp.dot(h, wc_ref[...], preferred_element_type=_f32) + bc_ref[...]
    m = jnp.max(logits, axis=1, keepdims=True)
    lse = m + jnp.log(jnp.sum(jnp.exp(logits - m), axis=1, keepdims=True))
    out_ref[...] = logits - lse


def _tc_classifier(acc, den, b, Wc, bc):
    return pl.pallas_call(
        _classifier_body,
        grid=(N // _RB,),
        in_specs=[
            pl.BlockSpec((NC, _RB, D), lambda i: (0, i, 0)),
            pl.BlockSpec((_RB, NC), lambda i: (i, 0)),
            pl.BlockSpec((1, D), lambda i: (0, 0)),
            pl.BlockSpec((D, C), lambda i: (0, 0)),
            pl.BlockSpec((1, C), lambda i: (0, 0)),
        ],
        out_specs=pl.BlockSpec((_RB, C), lambda i: (i, 0)),
        out_shape=jax.ShapeDtypeStruct((N, C), _f32),
    )(acc, den, b, Wc, bc)


_MESH = plsc.VectorSubcoreMesh(
    core_axis_name="c", subcore_axis_name="s", num_cores=NC, num_subcores=NS
)


NGRP = (NCHUNK - 1) // 4
ZR = NP // NS


@functools.partial(
    pl.kernel,
    mesh=_MESH,
    compiler_params=pltpu.CompilerParams(needs_layout_passes=False),
    out_type=[
        jax.ShapeDtypeStruct((NC, NP, D), _f32),
        jax.ShapeDtypeStruct((NC, 1, NP), _f32),
    ],
    scratch_types=[
        [pltpu.VMEM((K,), jnp.int32)] * 4,
        [pltpu.VMEM((K,), jnp.int32)] * 4,
        [pltpu.VMEM((K,), _f32)] * 4,
        [pltpu.VMEM((K,), _f32)] * 4,
        [pltpu.VMEM((K,), _f32)] * 2,
        [pltpu.VMEM((K, D), _f32)] * 4,
        pltpu.VMEM((ZR,), _f32),
        pltpu.VMEM_SHARED((NP, D), _f32),
        pltpu.VMEM_SHARED((NP,), _f32),
        pltpu.VMEM_SHARED((N,), _f32),
        pltpu.VMEM_SHARED((N,), _f32),
        [pltpu.SemaphoreType.DMA] * 4,
        [pltpu.SemaphoreType.DMA] * 4,
        [pltpu.SemaphoreType.DMA] * 4,
        [pltpu.SemaphoreType.DMA] * 4,
        [pltpu.SemaphoreType.DMA] * 4,
        [pltpu.SemaphoreType.DMA] * 2,
        [pltpu.SemaphoreType.DMA] * 4,
    ],
)
def _sc_edges(src_hbm, dst_hbm, as_hbm, ad_hbm, h_hbm, acc_hbm, den_hbm,
              sbufs, dbufs, pav, pdv, pvs, bufs, zbuf, acc_sh, den_sh,
              as_sh, ad_sh, si, sd, spa, spd, sg, sp, ss):
    cid = lax.axis_index("c")
    sid = lax.axis_index("s")
    wid = sid * NC + cid
    ebase = pl.multiple_of(wid * EPW, 8)

    pltpu.async_copy(src_hbm.at[pl.ds(ebase, K)], sbufs[0], si[0])
    pltpu.async_copy(dst_hbm.at[pl.ds(ebase, K)], dbufs[0], sd[0])
    pltpu.async_copy(src_hbm.at[pl.ds(ebase + K, K)], sbufs[1], si[1])
    pltpu.async_copy(dst_hbm.at[pl.ds(ebase + K, K)], dbufs[1], sd[1])

    for t in range(ZR // 16):
        zbuf[pl.ds(t * 16, 16)] = jnp.zeros((16,), _f32)

    def _zrow(i, carry):
        for j in range(D // 16):
            bufs[0][i, pl.ds(j * 16, 16)] = jnp.zeros((16,), _f32)
        return carry
    lax.fori_loop(0, K, _zrow, 0)

    @pl.when(sid == 0)
    def _():
        pltpu.async_copy(as_hbm, as_sh, sg[2])

    @pl.when(sid == 1)
    def _():
        pltpu.async_copy(ad_hbm, ad_sh, sg[2])

    rbase = pl.multiple_of(sid * ROWS_PT, 8)
    for t in range(ROWS_PT // K):
        pltpu.async_copy(bufs[0], acc_sh.at[pl.ds(rbase + t * K, K)], sg[3])
    pltpu.sync_copy(zbuf, den_sh.at[pl.ds(rbase, ZR)])
    for t in range(ROWS_PT // K):
        pltpu.make_async_copy(bufs[0], acc_sh.at[pl.ds(rbase, K)],
                              sg[3]).wait()

    @pl.when((sid == 0) | (sid == 1))
    def _():
        pltpu.make_async_copy(as_hbm, as_sh, sg[2]).wait()
    plsc.subcore_barrier()

    def _issue_idx(c, slot):
        base = pl.multiple_of(ebase + c * K, 8)
        pltpu.async_copy(src_hbm.at[pl.ds(base, K)], sbufs[slot], si[slot])
        pltpu.async_copy(dst_hbm.at[pl.ds(base, K)], dbufs[slot], sd[slot])

    def _wait(src, dst, sem):
        pltpu.make_async_copy(src, dst, sem).wait()

    def _issue_gathers(slot):
        pltpu.async_copy(h_hbm.at[sbufs[slot]], bufs[slot], sg[slot])
        pltpu.async_copy(as_sh.at[sbufs[slot]], pav[slot], spa[slot])
        pltpu.async_copy(ad_sh.at[dbufs[slot]], pdv[slot], spd[slot])

    def _do_chunk(c, b, in_loop):
        rb = b % 2
        nslot = (b + 2) % 4
        gslot = (b + 1) % 4

        @pl.when(c >= 2)
        def _():
            _wait(pvs[rb], den_sh.at[dbufs[nslot]], sp[rb])
            _wait(bufs[nslot], acc_sh.at[dbufs[0]], ss[nslot])
        if in_loop:
            @pl.when(c + 2 < NCHUNK)
            def _():
                _issue_idx(c + 2, nslot)

            _wait(src_hbm.at[pl.ds(ebase, K)], sbufs[gslot], si[gslot])
            _wait(dst_hbm.at[pl.ds(ebase, K)], dbufs[gslot], sd[gslot])
            _issue_gathers(gslot)

        _wait(as_sh.at[sbufs[0]], pav[b], spa[b])
        _wait(ad_sh.at[dbufs[0]], pdv[b], spd[b])
        for j in range(K // 16):
            e = pav[b][pl.ds(j * 16, 16)] + pdv[b][pl.ds(j * 16, 16)]
            e = jnp.where(e >= 0.0, e, 0.2 * e)
            pvs[rb][pl.ds(j * 16, 16)] = jnp.exp(e)
        pltpu.async_copy(pvs[rb], den_sh.at[dbufs[b]], sp[rb], add=True)

        _wait(h_hbm.at[sbufs[b]], bufs[b], sg[b])

        def _scale(t, c2):
            p16 = pvs[rb][pl.ds(t * 16, 16)]
            for l in range(16):
                i = t * 16 + l
                ps = p16[l]
                for j in range(D // 16):
                    bufs[b][i, pl.ds(j * 16, 16)] = (
                        bufs[b][i, pl.ds(j * 16, 16)] * ps)
            return c2
        lax.fori_loop(0, K // 16, _scale, 0)

        pltpu.async_copy(bufs[b], acc_sh.at[dbufs[b]], ss[b], add=True)

    _wait(src_hbm.at[pl.ds(ebase, K)], sbufs[0], si[0])
    _wait(dst_hbm.at[pl.ds(ebase, K)], dbufs[0], sd[0])
    _issue_gathers(0)

    def _group(g, carry):
        for b in range(4):
            _do_chunk(g * 4 + b, b, True)
        return carry

    lax.fori_loop(0, NGRP, _group, 0)
    _do_chunk(NCHUNK - 1, 0, False)

    _wait(bufs[3], acc_sh.at[dbufs[0]], ss[3])
    _wait(bufs[0], acc_sh.at[dbufs[0]], ss[0])
    for rb in range(2):
        _wait(pvs[rb], den_sh.at[dbufs[0]], sp[rb])

    plsc.subcore_barrier()
    pltpu.sync_copy(den_sh.at[pl.ds(rbase, ZR)],
                    den_hbm.at[cid, 0, pl.ds(rbase, ZR)])
    pltpu.sync_copy(acc_sh.at[pl.ds(rbase, ROWS_PT)],
                    acc_hbm.at[cid, pl.ds(rbase, ROWS_PT)])


def kernel(x, edge_index, W1, a_src1, a_dst1, b1, W2, a_src2, a_dst2, b2,
           Wc, bc):
    src = edge_index[0]
    dst = edge_index[1]
    a21 = jnp.stack([a_src1, a_dst1], axis=1)
    a22 = jnp.stack([a_src2, a_dst2], axis=1)

    h1, sa1 = _tc_proj(x, W1, a21)
    acc1, den1 = _sc_edges(src, dst, sa1[:, 0], sa1[:, 1], h1)
    h2, sa2 = _tc_combine_proj(acc1, den1.reshape(NC, NP)[:, :N].T,
                               b1.reshape(1, D), W2, a22)
    acc2, den2 = _sc_edges(src, dst, sa2[:, 0], sa2[:, 1], h2)
    return _tc_classifier(acc2, den2.reshape(NC, NP)[:, :N].T,
                          b2.reshape(1, D), Wc, bc.reshape(1, C))
